# baseline (speedup 1.0000x reference)
import functools

import jax
import jax.numpy as jnp
from jax import lax
from jax.experimental import pallas as pl
from jax.experimental.pallas import tpu as pltpu
from jax.experimental.pallas import tpu_sc as plsc

BATCH = 16384
DIM = 128
KOUT = 5
NPAD = 16
NWORKERS = 32
BPW = BATCH // NWORKERS
LANES = 16


def _combo_body(l1_ref, attr_ref, w_ref, b_ref, out_ref):
    l1 = jnp.concatenate(
        [l1_ref[:], jnp.zeros((32 - l1_ref.shape[0], DIM), jnp.float32)], axis=0)
    w = jnp.concatenate(
        [w_ref[:], jnp.zeros((NPAD - KOUT, DIM), jnp.float32)], axis=0)
    bias = jnp.concatenate(
        [b_ref[:], jnp.zeros((1, NPAD - KOUT), jnp.float32)], axis=1)
    for a in range(2):
        x = jnp.maximum(l1 + attr_ref[a:a + 1, :], 0.0)
        y = lax.dot_general(x, w, (((1,), (1,)), ((), ())),
                            preferred_element_type=jnp.float32) + bias
        for s in range(4):
            row = jnp.concatenate(
                [y[8 * s + q:8 * s + q + 1, :] for q in range(8)], axis=1)
            out_ref[a * 4 + s:a * 4 + s + 1, :] = row


_combo_call = pl.pallas_call(
    _combo_body,
    out_shape=jax.ShapeDtypeStruct((8, 128), jnp.float32),
)


@functools.partial(
    pl.kernel,
    mesh=plsc.VectorSubcoreMesh(core_axis_name="c", subcore_axis_name="s"),
    compiler_params=pltpu.CompilerParams(needs_layout_passes=False, use_tc_tiling_on_sc=False),
    out_type=jax.ShapeDtypeStruct((BATCH, KOUT), jnp.float32),
    scratch_types=[
        pltpu.VMEM((8, 128), jnp.float32),
        pltpu.VMEM((BPW,), jnp.int32),
        pltpu.VMEM((BPW,), jnp.int32),
        pltpu.VMEM((BPW, KOUT), jnp.float32),
        pltpu.SemaphoreType.DMA,
    ],
)
def _sc_gather(pair_hbm, attr_hbm, combo_hbm, out_hbm,
               combo_v, pair_v, attr_v, out_v, sem):
    wid = lax.axis_index("s") * 2 + lax.axis_index("c")
    base = wid * BPW
    d0 = pltpu.async_copy(combo_hbm, combo_v, sem)
    d1 = pltpu.async_copy(pair_hbm.at[pl.ds(base, BPW)], pair_v, sem)
    d2 = pltpu.async_copy(attr_hbm.at[pl.ds(base, BPW)], attr_v, sem)
    d0.wait()
    d1.wait()
    d2.wait()

    @plsc.parallel_loop(0, BPW // LANES, 1, unroll=4)
    def _chunk(k):
        kl = k * LANES
        p = pair_v[pl.ds(kl, LANES)]
        a = attr_v[pl.ds(kl, LANES)]
        c = (a * 32 + p) * NPAD
        i = lax.iota(jnp.int32, LANES) + kl
        for j in range(KOUT):
            t = c + j
            v = plsc.load_gather(
                combo_v, [lax.shift_right_logical(t, 7), t & 127])
            plsc.store_scatter(out_v, [i, jnp.full((LANES,), j, jnp.int32)], v)

    pltpu.sync_copy(out_v, out_hbm.at[pl.ds(base, BPW), :])


def kernel(pair_int, attribute_index, l1_table, attr_table, W, b):
    pair_i = pair_int.astype(jnp.int32)
    attr_i = attribute_index.astype(jnp.int32)
    combo = _combo_call(l1_table, attr_table, W, b.reshape(1, KOUT))
    return _sc_gather(pair_i, attr_i, combo)


# speedup vs baseline: 1.1343x; 1.1343x over previous
import functools

import jax
import jax.numpy as jnp
from jax import lax
from jax.experimental import pallas as pl
from jax.experimental.pallas import tpu as pltpu
from jax.experimental.pallas import tpu_sc as plsc

BATCH = 16384
DIM = 128
KOUT = 5
NPAD = 16
NWORKERS = 32
BPW = BATCH // NWORKERS
LANES = 16


def _combo_body(l1_ref, attr_ref, w_ref, b_ref, out_ref):
    l1 = jnp.concatenate(
        [l1_ref[:], jnp.zeros((32 - l1_ref.shape[0], DIM), jnp.float32)], axis=0)
    w = jnp.concatenate(
        [w_ref[:], jnp.zeros((NPAD - KOUT, DIM), jnp.float32)], axis=0)
    bias = jnp.concatenate(
        [b_ref[:], jnp.zeros((1, NPAD - KOUT), jnp.float32)], axis=1)
    for a in range(2):
        x = jnp.maximum(l1 + attr_ref[a:a + 1, :], 0.0)
        y = lax.dot_general(x, w, (((1,), (1,)), ((), ())),
                            preferred_element_type=jnp.float32) + bias
        for s in range(4):
            row = jnp.concatenate(
                [y[8 * s + q:8 * s + q + 1, :] for q in range(8)], axis=1)
            out_ref[a * 4 + s:a * 4 + s + 1, :] = row


_combo_call = pl.pallas_call(
    _combo_body,
    out_shape=jax.ShapeDtypeStruct((8, 128), jnp.float32),
)


@functools.partial(
    pl.kernel,
    mesh=plsc.VectorSubcoreMesh(core_axis_name="c", subcore_axis_name="s"),
    compiler_params=pltpu.CompilerParams(needs_layout_passes=False),
    out_type=jax.ShapeDtypeStruct((BATCH, KOUT), jnp.float32),
    scratch_types=[
        pltpu.VMEM((8, 128), jnp.float32),
        pltpu.VMEM((BPW,), jnp.int32),
        pltpu.VMEM((BPW,), jnp.int32),
        pltpu.VMEM((BPW, KOUT), jnp.float32),
        pltpu.SemaphoreType.DMA,
    ],
)
def _sc_gather(pair_hbm, attr_hbm, combo_hbm, out_hbm,
               combo_v, pair_v, attr_v, out_v, sem):
    wid = lax.axis_index("s") * 2 + lax.axis_index("c")
    base = wid * BPW
    d0 = pltpu.async_copy(combo_hbm, combo_v, sem)
    d1 = pltpu.async_copy(pair_hbm.at[pl.ds(base, BPW)], pair_v, sem)
    d2 = pltpu.async_copy(attr_hbm.at[pl.ds(base, BPW)], attr_v, sem)
    d0.wait()
    d1.wait()
    d2.wait()

    @plsc.parallel_loop(0, BPW // LANES, 1, unroll=4)
    def _chunk(k):
        kl = k * LANES
        p = pair_v[pl.ds(kl, LANES)]
        a = attr_v[pl.ds(kl, LANES)]
        c = (a * 32 + p) * NPAD
        i = lax.iota(jnp.int32, LANES) + kl
        for j in range(KOUT):
            t = c + j
            v = plsc.load_gather(
                combo_v, [lax.shift_right_logical(t, 7), t & 127])
            plsc.store_scatter(out_v, [i, jnp.full((LANES,), j, jnp.int32)], v)

    pltpu.sync_copy(out_v, out_hbm.at[pl.ds(base, BPW), :])


def kernel(pair_int, attribute_index, l1_table, attr_table, W, b):
    pair_i = pair_int.astype(jnp.int32)
    attr_i = attribute_index.astype(jnp.int32)
    combo = _combo_call(l1_table, attr_table, W, b.reshape(1, KOUT))
    return _sc_gather(pair_i, attr_i, combo)


# speedup vs baseline: 1.1487x; 1.0127x over previous
import functools

import jax
import jax.numpy as jnp
from jax import lax
from jax.experimental import pallas as pl
from jax.experimental.pallas import tpu as pltpu
from jax.experimental.pallas import tpu_sc as plsc

BATCH = 16384
DIM = 128
KOUT = 5
NPAD = 16
NWORKERS = 32
BPW = BATCH // NWORKERS
LANES = 16


def _combo_body(l1_ref, attr_ref, w_ref, b_ref, out_ref):
    l1 = jnp.concatenate(
        [l1_ref[:], jnp.zeros((32 - l1_ref.shape[0], DIM), jnp.float32)], axis=0)
    w = jnp.concatenate(
        [w_ref[:], jnp.zeros((NPAD - KOUT, DIM), jnp.float32)], axis=0)
    bias = jnp.concatenate(
        [b_ref[:], jnp.zeros((1, NPAD - KOUT), jnp.float32)], axis=1)
    for a in range(2):
        x = jnp.maximum(l1 + attr_ref[a:a + 1, :], 0.0)
        y = lax.dot_general(x, w, (((1,), (1,)), ((), ())),
                            preferred_element_type=jnp.float32) + bias
        for s in range(4):
            row = jnp.concatenate(
                [y[8 * s + q:8 * s + q + 1, :] for q in range(8)], axis=1)
            out_ref[a * 4 + s:a * 4 + s + 1, :] = row


_combo_call = pl.pallas_call(
    _combo_body,
    out_shape=jax.ShapeDtypeStruct((8, 128), jnp.float32),
)


@functools.partial(
    pl.kernel,
    mesh=plsc.VectorSubcoreMesh(core_axis_name="c", subcore_axis_name="s"),
    compiler_params=pltpu.CompilerParams(needs_layout_passes=False),
    out_type=jax.ShapeDtypeStruct((BATCH, KOUT), jnp.float32),
    scratch_types=[
        pltpu.VMEM((8, 128), jnp.float32),
        pltpu.VMEM((BPW,), jnp.int32),
        pltpu.VMEM((BPW,), jnp.int32),
        pltpu.VMEM((BPW, KOUT), jnp.float32),
        pltpu.SemaphoreType.DMA,
    ],
)
def _sc_gather(pair_hbm, attr_hbm, combo_hbm, out_hbm,
               combo_v, pair_v, attr_v, out_v, sem):
    wid = lax.axis_index("s") * 2 + lax.axis_index("c")
    base = wid * BPW
    d0 = pltpu.async_copy(combo_hbm, combo_v, sem)
    d1 = pltpu.async_copy(pair_hbm.at[pl.ds(base, BPW)], pair_v, sem)
    d2 = pltpu.async_copy(attr_hbm.at[pl.ds(base, BPW)], attr_v, sem)
    d0.wait()
    d1.wait()
    d2.wait()

    def _chunk(k):
        kl = k * LANES
        p = pair_v[pl.ds(kl, LANES)]
        a = attr_v[pl.ds(kl, LANES)]
        c = (a * 32 + p) * NPAD
        i = lax.iota(jnp.int32, LANES) + kl
        for j in range(KOUT):
            t = c + j
            v = plsc.load_gather(
                combo_v, [lax.shift_right_logical(t, 7), t & 127])
            plsc.store_scatter(out_v, [i, jnp.full((LANES,), j, jnp.int32)], v)

    half = BPW // 2
    plsc.parallel_loop(0, BPW // LANES // 2, 1, unroll=4)(_chunk)
    da = pltpu.async_copy(out_v.at[pl.ds(0, half), :],
                          out_hbm.at[pl.ds(base, half), :], sem)
    plsc.parallel_loop(BPW // LANES // 2, BPW // LANES, 1, unroll=4)(_chunk)
    db = pltpu.async_copy(out_v.at[pl.ds(half, half), :],
                          out_hbm.at[pl.ds(base + half, half), :], sem)
    da.wait()
    db.wait()


def kernel(pair_int, attribute_index, l1_table, attr_table, W, b):
    pair_i = pair_int.astype(jnp.int32)
    attr_i = attribute_index.astype(jnp.int32)
    combo = _combo_call(l1_table, attr_table, W, b.reshape(1, KOUT))
    return _sc_gather(pair_i, attr_i, combo)


# speedup vs baseline: 1.1552x; 1.0056x over previous
import functools

import jax
import jax.numpy as jnp
from jax import lax
from jax.experimental import pallas as pl
from jax.experimental.pallas import tpu as pltpu
from jax.experimental.pallas import tpu_sc as plsc

BATCH = 16384
DIM = 128
KOUT = 5
NPAD = 16
NWORKERS = 32
BPW = BATCH // NWORKERS
LANES = 16


def _combo_body(l1_ref, attr_ref, w_ref, b_ref, out_ref):
    l1 = jnp.concatenate(
        [l1_ref[:], jnp.zeros((32 - l1_ref.shape[0], DIM), jnp.float32)], axis=0)
    w = jnp.concatenate(
        [w_ref[:], jnp.zeros((NPAD - KOUT, DIM), jnp.float32)], axis=0)
    bias = jnp.concatenate(
        [b_ref[:], jnp.zeros((1, NPAD - KOUT), jnp.float32)], axis=1)
    for a in range(2):
        x = jnp.maximum(l1 + attr_ref[a:a + 1, :], 0.0)
        y = lax.dot_general(x, w, (((1,), (1,)), ((), ())),
                            preferred_element_type=jnp.float32) + bias
        for s in range(4):
            row = jnp.concatenate(
                [y[8 * s + q:8 * s + q + 1, :] for q in range(8)], axis=1)
            out_ref[a * 4 + s:a * 4 + s + 1, :] = row


_combo_call = pl.pallas_call(
    _combo_body,
    out_shape=jax.ShapeDtypeStruct((8, 128), jnp.float32),
)


@functools.partial(
    pl.kernel,
    mesh=plsc.VectorSubcoreMesh(core_axis_name="c", subcore_axis_name="s"),
    compiler_params=pltpu.CompilerParams(needs_layout_passes=False),
    out_type=jax.ShapeDtypeStruct((BATCH, KOUT), jnp.float32),
    scratch_types=[
        pltpu.VMEM((8, 128), jnp.float32),
        pltpu.VMEM((BPW,), jnp.int32),
        pltpu.VMEM((BPW,), jnp.int32),
        pltpu.VMEM((BPW, KOUT), jnp.float32),
        pltpu.SemaphoreType.DMA,
    ],
)
def _sc_gather(pair_hbm, attr_hbm, combo_hbm, out_hbm,
               combo_v, pair_v, attr_v, out_v, sem):
    wid = lax.axis_index("s") * 2 + lax.axis_index("c")
    base = wid * BPW
    d0 = pltpu.async_copy(combo_hbm, combo_v, sem)
    d1 = pltpu.async_copy(pair_hbm.at[pl.ds(base, BPW)], pair_v, sem)
    d2 = pltpu.async_copy(attr_hbm.at[pl.ds(base, BPW)], attr_v, sem)
    d0.wait()
    d1.wait()
    d2.wait()

    def _chunk(k):
        kl = k * LANES
        p = pair_v[pl.ds(kl, LANES)]
        a = attr_v[pl.ds(kl, LANES)]
        c = (a * 32 + p) * NPAD
        i = lax.iota(jnp.int32, LANES) + kl
        for j in range(KOUT):
            t = c + j
            v = plsc.load_gather(
                combo_v, [lax.shift_right_logical(t, 7), t & 127])
            plsc.store_scatter(out_v, [i, jnp.full((LANES,), j, jnp.int32)], v)

    half = BPW // 2
    plsc.parallel_loop(0, BPW // LANES // 2, 1, unroll=8)(_chunk)
    da = pltpu.async_copy(out_v.at[pl.ds(0, half), :],
                          out_hbm.at[pl.ds(base, half), :], sem)
    plsc.parallel_loop(BPW // LANES // 2, BPW // LANES, 1, unroll=8)(_chunk)
    db = pltpu.async_copy(out_v.at[pl.ds(half, half), :],
                          out_hbm.at[pl.ds(base + half, half), :], sem)
    da.wait()
    db.wait()


def kernel(pair_int, attribute_index, l1_table, attr_table, W, b):
    pair_i = pair_int.astype(jnp.int32)
    attr_i = attribute_index.astype(jnp.int32)
    combo = _combo_call(l1_table, attr_table, W, b.reshape(1, KOUT))
    return _sc_gather(pair_i, attr_i, combo)
